# baseline (speedup 1.0000x reference)
import functools

import jax
import jax.numpy as jnp
from jax.experimental import pallas as pl

FEAT = 512
WIN = 12
NOFF = 23
RAD = 11
SUB = 256
NSUB = 4
TILE = SUB * NSUB
SHALO = SUB + 24


def _dot(a, b):
    return jax.lax.dot_general(
        a, b, (((1,), (1,)), ((), ())), preferred_element_type=jnp.float32
    )


def _gru_kernel(x_ref, wih_ref, whh_ref, bih_ref, bhh_ref, o_ref, *, t_total):
    j = pl.program_id(1)
    base = j * TILE
    D = FEAT

    wih = wih_ref[...]
    whh = whh_ref[...].astype(jnp.bfloat16)
    bih = bih_ref[...]
    bhh = bhh_ref[...]
    gbias = jnp.concatenate(
        [bih[:, :2 * D] + bhh[:, :2 * D], bih[:, 2 * D:]], axis=1)
    bhh_n = bhh[:, 2 * D:].astype(jnp.bfloat16)

    row = jax.lax.broadcasted_iota(jnp.int32, (SUB, SHALO), 0)
    col = jax.lax.broadcasted_iota(jnp.int32, (SUB, SHALO), 1)
    o23 = jax.lax.broadcasted_iota(jnp.int32, (SUB, NOFF), 1)
    r23 = jax.lax.broadcasted_iota(jnp.int32, (SUB, NOFF), 0)
    a23 = jax.lax.broadcasted_iota(jnp.int32, (NOFF, NOFF), 0)
    b23 = jax.lax.broadcasted_iota(jnp.int32, (NOFF, NOFF), 1)
    ltri = (a23 < b23).astype(jnp.float32)
    off_f = o23.astype(jnp.float32)

    centers = []
    Gs = []
    sels = []
    ords = []
    for s in range(NSUB):
        sbase = base + s * SUB
        halo = x_ref[0, pl.ds(sbase, SHALO), :]
        center = halo[RAD:RAD + SUB, :]
        centers.append(center)
        S = _dot(center, halo)
        cols = []
        for o in range(NOFF):
            m = col == row + o
            cols.append(jnp.sum(jnp.where(m, S, 0.0), axis=1, keepdims=True))
        Sb = jnp.concatenate(cols, axis=1)
        nbr = sbase + r23 + o23 - RAD
        Sb = jnp.where((nbr >= 0) & (nbr < t_total), Sb, -1e9)

        keep = jnp.ones((SUB, NOFF), jnp.bool_)
        for _ in range(NOFF - WIN):
            m = jnp.min(Sb, axis=1, keepdims=True)
            eq = Sb == m
            last = jnp.max(jnp.where(eq, o23, -1), axis=1, keepdims=True)
            oh = o23 == last
            keep = keep & ~oh
            Sb = jnp.where(oh, jnp.inf, Sb)
        sels.append(keep)
        ords.append(jax.lax.dot_general(
            keep.astype(jnp.float32), ltri, (((1,), (0,)), ((), ())),
            preferred_element_type=jnp.float32))
        Gs.append((_dot(halo, wih) + gbias).astype(jnp.bfloat16))

    hs = [jnp.zeros((SUB, D), jnp.float32) for _ in range(NSUB)]
    for w in range(WIN):
        for s in range(NSUB):
            ohw = jnp.where(sels[s] & (ords[s] == float(w)), 1.0, 0.0)
            off = jnp.sum(ohw * off_f, axis=1, keepdims=True).astype(jnp.int32)
            P = (col == row + off).astype(jnp.bfloat16)
            gi = jax.lax.dot_general(
                P, Gs[s], (((1,), (0,)), ((), ())),
                preferred_element_type=jnp.float32).astype(jnp.bfloat16)
            gh = _dot(hs[s].astype(jnp.bfloat16), whh).astype(jnp.bfloat16)
            r = jax.nn.sigmoid(gi[:, :D] + gh[:, :D])
            z = jax.nn.sigmoid(gi[:, D:2 * D] + gh[:, D:2 * D])
            n = jnp.tanh(gi[:, 2 * D:] + r * (gh[:, 2 * D:] + bhh_n))
            nf = n.astype(jnp.float32)
            hs[s] = nf + z.astype(jnp.float32) * (hs[s] - nf)

    for s in range(NSUB):
        o_ref[0, s * SUB:(s + 1) * SUB, :] = hs[s] + centers[s]


def kernel(x, w_ih, w_hh, b_ih, b_hh):
    B, T, D = x.shape
    nt = T // TILE
    pad_r = (SHALO - SUB) - RAD
    x_pad = jnp.pad(x, ((0, 0), (RAD, pad_r), (0, 0)))
    kern = functools.partial(_gru_kernel, t_total=T)
    out = pl.pallas_call(
        kern,
        grid=(B, nt),
        in_specs=[
            pl.BlockSpec((1, T + (SHALO - SUB), D), lambda b, j: (b, 0, 0)),
            pl.BlockSpec((3 * D, D), lambda b, j: (0, 0)),
            pl.BlockSpec((3 * D, D), lambda b, j: (0, 0)),
            pl.BlockSpec((1, 3 * D), lambda b, j: (0, 0)),
            pl.BlockSpec((1, 3 * D), lambda b, j: (0, 0)),
        ],
        out_specs=pl.BlockSpec((1, TILE, D), lambda b, j: (b, j, 0)),
        out_shape=jax.ShapeDtypeStruct((B, T, D), x.dtype),
    )(x_pad, w_ih, w_hh, b_ih.reshape(1, -1), b_hh.reshape(1, -1))
    return out


# speedup vs baseline: 1.0012x; 1.0012x over previous
import functools

import jax
import jax.numpy as jnp
from jax.experimental import pallas as pl
from jax.experimental.pallas import tpu as pltpu

FEAT = 512
WIN = 12
NOFF = 23
RAD = 11
TILE = 256
SHALO = TILE + 24


def _dot(a, b):
    return jax.lax.dot_general(
        a, b, (((1,), (1,)), ((), ())), preferred_element_type=jnp.float32
    )


def _gru_kernel(x_ref, wih_ref, whh_ref, bih_ref, bhh_ref, o_ref,
                g_sc, c_sc, sel_sc, ord_sc, *, t_total, nt):
    j = pl.program_id(1)
    D = FEAT

    row = jax.lax.broadcasted_iota(jnp.int32, (TILE, SHALO), 0)
    col = jax.lax.broadcasted_iota(jnp.int32, (TILE, SHALO), 1)
    o23 = jax.lax.broadcasted_iota(jnp.int32, (TILE, NOFF), 1)

    @pl.when(j < nt)
    def _phase1():
        base = j * TILE
        halo = x_ref[0, pl.ds(base, SHALO), :]
        center = halo[RAD:RAD + TILE, :]
        S = _dot(center, halo)
        cols = []
        for o in range(NOFF):
            m = col == row + o
            cols.append(jnp.sum(jnp.where(m, S, 0.0), axis=1, keepdims=True))
        Sb = jnp.concatenate(cols, axis=1)
        r23 = jax.lax.broadcasted_iota(jnp.int32, (TILE, NOFF), 0)
        nbr = base + r23 + o23 - RAD
        Sb = jnp.where((nbr >= 0) & (nbr < t_total), Sb, -1e9)

        keep = jnp.ones((TILE, NOFF), jnp.bool_)
        Sw = Sb
        for _ in range(NOFF - WIN):
            m = jnp.min(Sw, axis=1, keepdims=True)
            eq = Sw == m
            last = jnp.max(jnp.where(eq, o23, -1), axis=1, keepdims=True)
            oh = o23 == last
            keep = keep & ~oh
            Sw = jnp.where(oh, jnp.inf, Sw)
        keep_f = keep.astype(jnp.float32)

        a23 = jax.lax.broadcasted_iota(jnp.int32, (NOFF, NOFF), 0)
        b23 = jax.lax.broadcasted_iota(jnp.int32, (NOFF, NOFF), 1)
        ltri = (a23 < b23).astype(jnp.float32)
        ordv = jax.lax.dot_general(
            keep_f, ltri, (((1,), (0,)), ((), ())),
            preferred_element_type=jnp.float32)

        wih = wih_ref[...]
        bih = bih_ref[...]
        bhh = bhh_ref[...]
        gbias = jnp.concatenate(
            [bih[:, :2 * D] + bhh[:, :2 * D], bih[:, 2 * D:]], axis=1)
        slot = jax.lax.rem(j, 2)
        g_sc[slot] = (_dot(halo, wih) + gbias).astype(jnp.bfloat16)
        c_sc[slot] = center
        sel_sc[slot] = keep_f
        ord_sc[slot] = ordv

    @pl.when(j > 0)
    def _phase2():
        jj = j - 1
        slot = jax.lax.rem(jj, 2)
        G = g_sc[slot]
        center = c_sc[slot]
        sel = sel_sc[slot] == 1.0
        ordv = ord_sc[slot]
        whh = whh_ref[...].astype(jnp.bfloat16)
        bhh_n = bhh_ref[:, 2 * D:].astype(jnp.bfloat16)
        off_f = o23.astype(jnp.float32)

        h = jnp.zeros((TILE, D), jnp.float32)
        for w in range(WIN):
            ohw = jnp.where(sel & (ordv == float(w)), 1.0, 0.0)
            off = jnp.sum(ohw * off_f, axis=1,
                          keepdims=True).astype(jnp.int32)
            P = (col == row + off).astype(jnp.bfloat16)
            gi = jax.lax.dot_general(
                P, G, (((1,), (0,)), ((), ())),
                preferred_element_type=jnp.float32).astype(jnp.bfloat16)
            gh = _dot(h.astype(jnp.bfloat16), whh).astype(jnp.bfloat16)
            r = jax.nn.sigmoid(gi[:, :D] + gh[:, :D])
            z = jax.nn.sigmoid(gi[:, D:2 * D] + gh[:, D:2 * D])
            n = jnp.tanh(gi[:, 2 * D:] + r * (gh[:, 2 * D:] + bhh_n))
            nf = n.astype(jnp.float32)
            h = nf + z.astype(jnp.float32) * (h - nf)

        o_ref[0, :, :] = h + center


def kernel(x, w_ih, w_hh, b_ih, b_hh):
    B, T, D = x.shape
    nt = T // TILE
    pad_r = (SHALO - TILE) - RAD
    x_pad = jnp.pad(x, ((0, 0), (RAD, pad_r), (0, 0)))
    kern = functools.partial(_gru_kernel, t_total=T, nt=nt)
    out = pl.pallas_call(
        kern,
        grid=(B, nt + 1),
        in_specs=[
            pl.BlockSpec((1, T + (SHALO - TILE), D), lambda b, j: (b, 0, 0)),
            pl.BlockSpec((3 * D, D), lambda b, j: (0, 0)),
            pl.BlockSpec((3 * D, D), lambda b, j: (0, 0)),
            pl.BlockSpec((1, 3 * D), lambda b, j: (0, 0)),
            pl.BlockSpec((1, 3 * D), lambda b, j: (0, 0)),
        ],
        out_specs=pl.BlockSpec(
            (1, TILE, D),
            lambda b, j: (b, jnp.maximum(j - 1, 0), 0)),
        out_shape=jax.ShapeDtypeStruct((B, T, D), x.dtype),
        scratch_shapes=[
            pltpu.VMEM((2, SHALO, 3 * D), jnp.bfloat16),
            pltpu.VMEM((2, TILE, D), jnp.float32),
            pltpu.VMEM((2, TILE, NOFF), jnp.float32),
            pltpu.VMEM((2, TILE, NOFF), jnp.float32),
        ],
    )(x_pad, w_ih, w_hh, b_ih.reshape(1, -1), b_hh.reshape(1, -1))
    return out


# speedup vs baseline: 1.1468x; 1.1454x over previous
import functools

import jax
import jax.numpy as jnp
from jax.experimental import pallas as pl
from jax.experimental.pallas import tpu as pltpu

FEAT = 512
WIN = 12
NOFF = 23
RAD = 11
TILE = 256
SHALO = TILE + 24


def _dot(a, b):
    return jax.lax.dot_general(
        a, b, (((1,), (1,)), ((), ())), preferred_element_type=jnp.float32
    )


def _gru_kernel(x_ref, wih_ref, whh_ref, bih_ref, bhh_ref, o_ref, *, t_total):
    j = pl.program_id(1)
    base = j * TILE
    D = FEAT

    wih = wih_ref[...]
    whh = whh_ref[...].astype(jnp.bfloat16)
    bih = bih_ref[...]
    bhh = bhh_ref[...]
    gbias = jnp.concatenate(
        [bih[:, :2 * D] + bhh[:, :2 * D], bih[:, 2 * D:]], axis=1)
    bhh_n = bhh[:, 2 * D:].astype(jnp.bfloat16)

    halo = x_ref[0, pl.ds(base, SHALO), :]
    center = halo[RAD:RAD + TILE, :]

    S = _dot(center, halo)
    row = jax.lax.broadcasted_iota(jnp.int32, (TILE, SHALO), 0)
    col = jax.lax.broadcasted_iota(jnp.int32, (TILE, SHALO), 1)
    cols = []
    for o in range(NOFF):
        m = col == row + o
        cols.append(jnp.sum(jnp.where(m, S, 0.0), axis=1, keepdims=True))
    Sb = jnp.concatenate(cols, axis=1)

    r23 = jax.lax.broadcasted_iota(jnp.int32, (TILE, NOFF), 0)
    o23 = jax.lax.broadcasted_iota(jnp.int32, (TILE, NOFF), 1)
    nbr = base + r23 + o23 - RAD
    Sb = jnp.where((nbr >= 0) & (nbr < t_total), Sb, -1e9)

    keep = jnp.ones((TILE, NOFF), jnp.bool_)
    Sw = Sb
    for _ in range(NOFF - WIN):
        m = jnp.min(Sw, axis=1, keepdims=True)
        eq = Sw == m
        last = jnp.max(jnp.where(eq, o23, -1), axis=1, keepdims=True)
        oh = o23 == last
        keep = keep & ~oh
        Sw = jnp.where(oh, jnp.inf, Sw)
    sel = keep
    sel_f = sel.astype(jnp.float32)

    a23 = jax.lax.broadcasted_iota(jnp.int32, (NOFF, NOFF), 0)
    b23 = jax.lax.broadcasted_iota(jnp.int32, (NOFF, NOFF), 1)
    ltri = (a23 < b23).astype(jnp.float32)
    ordv = jax.lax.dot_general(
        sel_f, ltri, (((1,), (0,)), ((), ())),
        preferred_element_type=jnp.float32)

    G = (_dot(halo, wih) + gbias).astype(jnp.bfloat16)

    h = jnp.zeros((TILE, D), jnp.float32)
    off_f = o23.astype(jnp.float32)
    for w in range(WIN):
        ohw = jnp.where(sel & (ordv == float(w)), 1.0, 0.0)
        off = jnp.sum(ohw * off_f, axis=1, keepdims=True).astype(jnp.int32)
        P = (col == row + off).astype(jnp.bfloat16)
        gi = jax.lax.dot_general(
            P, G, (((1,), (0,)), ((), ())),
            preferred_element_type=jnp.float32).astype(jnp.bfloat16)
        gh = _dot(h.astype(jnp.bfloat16), whh).astype(jnp.bfloat16)
        r = jax.nn.sigmoid(gi[:, :D] + gh[:, :D])
        z = jax.nn.sigmoid(gi[:, D:2 * D] + gh[:, D:2 * D])
        n = jnp.tanh(gi[:, 2 * D:] + r * (gh[:, 2 * D:] + bhh_n))
        nf = n.astype(jnp.float32)
        h = nf + z.astype(jnp.float32) * (h - nf)

    o_ref[0, :, :] = h + center


def kernel(x, w_ih, w_hh, b_ih, b_hh):
    B, T, D = x.shape
    nt = T // TILE
    pad_r = (SHALO - TILE) - RAD
    x_pad = jnp.pad(x, ((0, 0), (RAD, pad_r), (0, 0)))
    kern = functools.partial(_gru_kernel, t_total=T)
    out = pl.pallas_call(
        kern,
        grid=(B, nt),
        in_specs=[
            pl.BlockSpec((1, T + (SHALO - TILE), D), lambda b, j: (b, 0, 0)),
            pl.BlockSpec((3 * D, D), lambda b, j: (0, 0)),
            pl.BlockSpec((3 * D, D), lambda b, j: (0, 0)),
            pl.BlockSpec((1, 3 * D), lambda b, j: (0, 0)),
            pl.BlockSpec((1, 3 * D), lambda b, j: (0, 0)),
        ],
        out_specs=pl.BlockSpec((1, TILE, D), lambda b, j: (b, j, 0)),
        out_shape=jax.ShapeDtypeStruct((B, T, D), x.dtype),
        compiler_params=pltpu.CompilerParams(
            dimension_semantics=("parallel", "parallel")),
    )(x_pad, w_ih, w_hh, b_ih.reshape(1, -1), b_hh.reshape(1, -1))
    return out


# speedup vs baseline: 1.1490x; 1.0020x over previous
import functools

import jax
import jax.numpy as jnp
from jax.experimental import pallas as pl


FEAT = 512
WIN = 12
NOFF = 23
RAD = 11
TILE = 256
SHALO = TILE + 24


def _dot(a, b):
    return jax.lax.dot_general(
        a, b, (((1,), (1,)), ((), ())), preferred_element_type=jnp.float32
    )


def _gru_kernel(x_ref, wih_ref, whh_ref, bih_ref, bhh_ref, o_ref, *, t_total):
    j = pl.program_id(1)
    base = j * TILE
    D = FEAT

    wih = wih_ref[...]
    whh = whh_ref[...].astype(jnp.bfloat16)
    bih = bih_ref[...]
    bhh = bhh_ref[...]
    gbias = jnp.concatenate(
        [bih[:, :2 * D] + bhh[:, :2 * D], bih[:, 2 * D:]], axis=1)
    bhh_n = bhh[:, 2 * D:].astype(jnp.bfloat16)

    halo = x_ref[0, pl.ds(base, SHALO), :]
    center = halo[RAD:RAD + TILE, :]

    S = _dot(center, halo)
    row = jax.lax.broadcasted_iota(jnp.int32, (TILE, SHALO), 0)
    col = jax.lax.broadcasted_iota(jnp.int32, (TILE, SHALO), 1)
    cols = []
    for o in range(NOFF):
        m = col == row + o
        cols.append(jnp.sum(jnp.where(m, S, 0.0), axis=1, keepdims=True))
    Sb = jnp.concatenate(cols, axis=1)

    r23 = jax.lax.broadcasted_iota(jnp.int32, (TILE, NOFF), 0)
    o23 = jax.lax.broadcasted_iota(jnp.int32, (TILE, NOFF), 1)
    nbr = base + r23 + o23 - RAD
    Sb = jnp.where((nbr >= 0) & (nbr < t_total), Sb, -1e9)

    keep = jnp.ones((TILE, NOFF), jnp.bool_)
    Sw = Sb
    for _ in range(NOFF - WIN):
        m = jnp.min(Sw, axis=1, keepdims=True)
        eq = Sw == m
        last = jnp.max(jnp.where(eq, o23, -1), axis=1, keepdims=True)
        oh = o23 == last
        keep = keep & ~oh
        Sw = jnp.where(oh, jnp.inf, Sw)
    sel = keep
    sel_f = sel.astype(jnp.float32)

    a23 = jax.lax.broadcasted_iota(jnp.int32, (NOFF, NOFF), 0)
    b23 = jax.lax.broadcasted_iota(jnp.int32, (NOFF, NOFF), 1)
    ltri = (a23 < b23).astype(jnp.float32)
    ordv = jax.lax.dot_general(
        sel_f, ltri, (((1,), (0,)), ((), ())),
        preferred_element_type=jnp.float32)

    G = (_dot(halo, wih) + gbias).astype(jnp.bfloat16)

    h = jnp.zeros((TILE, D), jnp.float32)
    off_f = o23.astype(jnp.float32)
    for w in range(WIN):
        ohw = jnp.where(sel & (ordv == float(w)), 1.0, 0.0)
        off = jnp.sum(ohw * off_f, axis=1, keepdims=True).astype(jnp.int32)
        P = (col == row + off).astype(jnp.bfloat16)
        gi = jax.lax.dot_general(
            P, G, (((1,), (0,)), ((), ())),
            preferred_element_type=jnp.float32).astype(jnp.bfloat16)
        gh = _dot(h.astype(jnp.bfloat16), whh).astype(jnp.bfloat16)
        r = jax.nn.sigmoid(gi[:, :D] + gh[:, :D])
        z = jax.nn.sigmoid(gi[:, D:2 * D] + gh[:, D:2 * D])
        n = jnp.tanh(gi[:, 2 * D:] + r * (gh[:, 2 * D:] + bhh_n))
        nf = n.astype(jnp.float32)
        h = nf + z.astype(jnp.float32) * (h - nf)

    o_ref[0, :, :] = h + center


def kernel(x, w_ih, w_hh, b_ih, b_hh):
    B, T, D = x.shape
    nt = T // TILE
    pad_r = (SHALO - TILE) - RAD
    x_pad = jnp.pad(x, ((0, 0), (RAD, pad_r), (0, 0)))
    kern = functools.partial(_gru_kernel, t_total=T)
    out = pl.pallas_call(
        kern,
        grid=(B, nt),
        in_specs=[
            pl.BlockSpec((1, T + (SHALO - TILE), D), lambda b, j: (b, 0, 0)),
            pl.BlockSpec((3 * D, D), lambda b, j: (0, 0)),
            pl.BlockSpec((3 * D, D), lambda b, j: (0, 0)),
            pl.BlockSpec((1, 3 * D), lambda b, j: (0, 0)),
            pl.BlockSpec((1, 3 * D), lambda b, j: (0, 0)),
        ],
        out_specs=pl.BlockSpec((1, TILE, D), lambda b, j: (b, j, 0)),
        out_shape=jax.ShapeDtypeStruct((B, T, D), x.dtype),
    )(x_pad, w_ih, w_hh, b_ih.reshape(1, -1), b_hh.reshape(1, -1))
    return out
